# Initial kernel scaffold; baseline (speedup 1.0000x reference)
import jax
import jax.numpy as jnp
from jax.experimental import pallas as pl


def kernel(x, edge_index, Wl1, bl1, Wr1, br1, att1, bias1, Wlin1, blin1, Wl2, bl2, Wr2, br2, att2, bias2, Wlin2, blin2):
    raise NotImplementedError("write your pallas kernel here")


# speedup vs baseline: 4.8210x; 4.8210x over previous
import jax
import jax.numpy as jnp
from jax import lax
from jax.experimental import pallas as pl
from jax.experimental.pallas import tpu as pltpu
from jax.experimental.pallas import tpu_sc as plsc

N = 10000
E = 320000
FIN = 128
H = 128
C = 40

NPAD = 10240
NSC = 10112
DUMMY = N
B = 128
NT = 32
E1 = E + N
CHUNKS = -(-E1 // (NT * B))
EPT = CHUNKS * B
EPAD = EPT * NT
GROWS = EPAD // B
NR = NSC // 16
BM = 1024
NJ = H // 16


def _mm3_body(x_ref, wa, ba, wb, bb, wc, bc, oa, ob, oc):
    xv = x_ref[...]
    oa[...] = jnp.dot(xv, wa[...], preferred_element_type=jnp.float32) + ba[...]
    ob[...] = jnp.dot(xv, wb[...], preferred_element_type=jnp.float32) + bb[...]
    oc[...] = jnp.dot(xv, wc[...], preferred_element_type=jnp.float32) + bc[...]


def _mm3(x, wa, ba, wb, bb, wc, bc):
    k = x.shape[1]
    d = wa.shape[1]
    row = pl.BlockSpec((BM, k), lambda i: (i, 0))
    full_w = pl.BlockSpec((k, d), lambda i: (0, 0))
    full_b = pl.BlockSpec((1, d), lambda i: (0, 0))
    out = pl.BlockSpec((BM, d), lambda i: (i, 0))
    o = jax.ShapeDtypeStruct((NPAD, d), jnp.float32)
    return pl.pallas_call(
        _mm3_body,
        grid=(NPAD // BM,),
        in_specs=[row, full_w, full_b, full_w, full_b, full_w, full_b],
        out_specs=[out, out, out],
        out_shape=[o, o, o],
    )(x, wa, ba.reshape(1, d), wb, bb.reshape(1, d), wc, bc.reshape(1, d))


def _max_body(al_ref, out_ref):
    out_ref[...] = jnp.broadcast_to(
        jnp.max(al_ref[...], axis=(0, 2)).reshape(1, 1, H), (1, 8, H))


def _tc_blockmax(al3):
    return pl.pallas_call(
        _max_body,
        grid=(GROWS // 32,),
        in_specs=[pl.BlockSpec((32, H, 16), lambda i: (i, 0, 0))],
        out_specs=pl.BlockSpec((1, 8, H), lambda i: (i, 0, 0)),
        out_shape=jax.ShapeDtypeStruct((GROWS // 32, 8, H), jnp.float32),
    )(al3)


def _exp_body(al_ref, bm_ref, out_ref):
    m = jnp.max(bm_ref[...])
    a = jnp.sum(al_ref[...], axis=2)
    out_ref[...] = jnp.broadcast_to(
        jnp.exp(a - m)[:, :, None], (32, H, 16))


def _tc_exp(al3, bmax):
    return pl.pallas_call(
        _exp_body,
        grid=(GROWS // 32,),
        in_specs=[pl.BlockSpec((32, H, 16), lambda i: (i, 0, 0)),
                  pl.BlockSpec((GROWS // 32, 8, H), lambda i: (0, 0, 0))],
        out_specs=pl.BlockSpec((32, H, 16), lambda i: (i, 0, 0)),
        out_shape=jax.ShapeDtypeStruct((GROWS, H, 16), jnp.float32),
    )(al3, bmax)


def _mid_body(num_ref, den_ref, lin_ref, b1_ref, wl, bl, wr, br, wn, bn,
              oxl, oxr, olin):
    num = num_ref[0] + num_ref[1]
    den = jnp.sum(den_ref[...], axis=0).reshape(-1, 1)
    conv = jnp.where(den > 0, num / den, 0.0) + b1_ref[...]
    pre = conv + lin_ref[...]
    h = jnp.where(pre > 0, pre, jnp.exp(pre) - 1.0)
    oxl[...] = jnp.dot(h, wl[...], preferred_element_type=jnp.float32) + bl[...]
    oxr[...] = jnp.dot(h, wr[...], preferred_element_type=jnp.float32) + br[...]
    olin[...] = jnp.dot(h, wn[...], preferred_element_type=jnp.float32) + bn[...]


def _tc_mid(num, den, lin1, b1, wl, bl, wr, br, wn, bn):
    full_w = pl.BlockSpec((H, H), lambda i: (0, 0))
    full_b = pl.BlockSpec((1, H), lambda i: (0, 0))
    o = jax.ShapeDtypeStruct((NPAD, H), jnp.float32)
    return pl.pallas_call(
        _mid_body,
        grid=(NPAD // BM,),
        in_specs=[
            pl.BlockSpec((2, BM, H), lambda i: (0, i, 0)),
            pl.BlockSpec((NT, BM), lambda i: (0, i)),
            pl.BlockSpec((BM, H), lambda i: (i, 0)),
            full_b, full_w, full_b, full_w, full_b, full_w, full_b,
        ],
        out_specs=[pl.BlockSpec((BM, H), lambda i: (i, 0))] * 3,
        out_shape=[o, o, o],
    )(num, den, lin1, b1, wl, bl, wr, br, wn, bn)


def _post_body(num_ref, den_ref, lin_ref, b2_ref, out_ref):
    num = num_ref[0] + num_ref[1]
    den = jnp.sum(den_ref[...], axis=0).reshape(-1, 1)
    o = jnp.where(den > 0, num / den, 0.0) + b2_ref[...] + lin_ref[...]
    col = lax.broadcasted_iota(jnp.int32, (BM, H), 1)
    mask = col < C
    om = jnp.where(mask, o, -1e30)
    m = jnp.max(om, axis=1, keepdims=True)
    ex = jnp.where(mask, jnp.exp(om - m), 0.0)
    s = jnp.sum(ex, axis=1, keepdims=True)
    out_ref[...] = o - m - jnp.log(s)


def _tc_post(num2, den2, lin2, b2):
    return pl.pallas_call(
        _post_body,
        grid=(NPAD // BM,),
        in_specs=[
            pl.BlockSpec((2, BM, H), lambda i: (0, i, 0)),
            pl.BlockSpec((NT, BM), lambda i: (0, i)),
            pl.BlockSpec((BM, H), lambda i: (i, 0)),
            pl.BlockSpec((1, H), lambda i: (0, 0)),
        ],
        out_specs=pl.BlockSpec((BM, H), lambda i: (i, 0)),
        out_shape=jax.ShapeDtypeStruct((NPAD, H), jnp.float32),
    )(num2, den2, lin2, b2)


def _sc_alpha_body(xl_hbm, xr_hbm, src_hbm, dst_hbm, att_hbm,
                   al_hbm,
                   src_v, dst_v, xl_rows, xr_rows, att_v, al_buf):
    cid = lax.axis_index("c")
    sid = lax.axis_index("s")
    wid = cid * 16 + sid
    pltpu.sync_copy(att_hbm, att_v)
    attv = [att_v[pl.ds(16 * j, 16)] for j in range(NJ)]

    def chunk(c, carry):
        base = wid * EPT + c * B
        pltpu.sync_copy(src_hbm.at[pl.ds(base, B)], src_v)
        pltpu.sync_copy(dst_hbm.at[pl.ds(base, B)], dst_v)
        pltpu.sync_copy(xl_hbm.at[src_v], xl_rows)
        pltpu.sync_copy(xr_hbm.at[dst_v], xr_rows)

        def e_body(e, ec):
            acc = jnp.zeros((16,), jnp.float32)
            for j in range(NJ):
                a = xl_rows[e, pl.ds(16 * j, 16)]
                b = xr_rows[e, pl.ds(16 * j, 16)]
                s = a + b
                acc = acc + jnp.maximum(s, 0.2 * s) * attv[j]
            al_buf[pl.ds(e * 16, 16)] = acc
            return ec

        lax.fori_loop(0, B, e_body, 0)
        pltpu.sync_copy(al_buf, al_hbm.at[pl.ds(base * 16, B * 16)])
        return carry

    lax.fori_loop(0, CHUNKS, chunk, 0)


_sc_alpha = pl.kernel(
    _sc_alpha_body,
    mesh=plsc.VectorSubcoreMesh(core_axis_name="c", subcore_axis_name="s"),
    out_type=jax.ShapeDtypeStruct((EPAD * 16,), jnp.float32),
    scratch_types=[
        pltpu.VMEM((B,), jnp.int32),
        pltpu.VMEM((B,), jnp.int32),
        pltpu.VMEM((B, H), jnp.float32),
        pltpu.VMEM((B, H), jnp.float32),
        pltpu.VMEM((H,), jnp.float32),
        pltpu.VMEM((B * 16,), jnp.float32),
    ],
)


def _sc_scatter_body(xl_hbm, src_hbm, dst_hbm, ex_hbm,
                     num_hbm, den_hbm,
                     src_v, dst_v, xl_rows, ex_buf, scaled, den_p, num_sh):
    cid = lax.axis_index("c")
    sid = lax.axis_index("s")
    wid = cid * 16 + sid
    rpt = NSC // 16
    iota = lax.iota(jnp.int32, 16)

    def zrow(r, carry):
        z = jnp.zeros((16,), jnp.float32)
        for j in range(NJ):
            scaled[r, pl.ds(16 * j, 16)] = z
        return carry

    lax.fori_loop(0, B, zrow, 0)

    def zden(r, carry):
        den_p[pl.ds(r * 16, 16)] = jnp.zeros((16,), jnp.float32)
        return carry

    lax.fori_loop(0, NR, zden, 0)
    for k in range(rpt // B):
        pltpu.sync_copy(scaled, num_sh.at[pl.ds(sid * rpt + k * B, B)])
    rem = rpt % B
    if rem:
        pltpu.sync_copy(scaled.at[pl.ds(0, rem)],
                        num_sh.at[pl.ds(sid * rpt + rpt - rem, rem)])
    plsc.subcore_barrier()

    def chunk(c, carry):
        base = wid * EPT + c * B
        pltpu.sync_copy(src_hbm.at[pl.ds(base, B)], src_v)
        pltpu.sync_copy(dst_hbm.at[pl.ds(base, B)], dst_v)
        pltpu.sync_copy(ex_hbm.at[pl.ds(base * 16, B * 16)], ex_buf)
        pltpu.sync_copy(xl_hbm.at[src_v], xl_rows)

        def g_body(g, ec):
            dvec = dst_v[pl.ds(g * 16, 16)]
            for i in range(16):
                e = g * 16 + i
                exw = ex_buf[pl.ds(e * 16, 16)]
                for j in range(NJ):
                    v = xl_rows[e, pl.ds(16 * j, 16)]
                    scaled[e, pl.ds(16 * j, 16)] = v * exw
                d = dvec[i]
                s0 = (d // 16) * 16
                lane = d - s0
                acc = den_p[pl.ds(s0, 16)]
                den_p[pl.ds(s0, 16)] = acc + jnp.where(
                    iota == lane, exw, 0.0)
            return ec

        lax.fori_loop(0, B // 16, g_body, 0)
        pltpu.sync_copy(scaled, num_sh.at[dst_v], add=True)
        return carry

    lax.fori_loop(0, CHUNKS, chunk, 0)
    pltpu.sync_copy(den_p, den_hbm.at[wid])
    plsc.subcore_barrier()
    r0 = sid * rpt
    pltpu.sync_copy(num_sh.at[pl.ds(r0, rpt)], num_hbm.at[cid, pl.ds(r0, rpt)])


_sc_scatter = pl.kernel(
    _sc_scatter_body,
    mesh=plsc.VectorSubcoreMesh(core_axis_name="c", subcore_axis_name="s"),
    out_type=[jax.ShapeDtypeStruct((2, NPAD, H), jnp.float32),
              jax.ShapeDtypeStruct((NT, NSC), jnp.float32)],
    scratch_types=[
        pltpu.VMEM((B,), jnp.int32),
        pltpu.VMEM((B,), jnp.int32),
        pltpu.VMEM((B, H), jnp.float32),
        pltpu.VMEM((B * 16,), jnp.float32),
        pltpu.VMEM((B, H), jnp.float32),
        pltpu.VMEM((NR * 16,), jnp.float32),
        pltpu.VMEM_SHARED((NSC, H), jnp.float32),
    ],
)


def _edge_softmax_scatter(xl, xr, src, dst, att):
    al = _sc_alpha(xl, xr, src, dst, att)
    al3 = al.reshape(GROWS, H, 16)
    bmax = _tc_blockmax(al3)
    ex = _tc_exp(al3, bmax).reshape(EPAD * 16)
    num, den = _sc_scatter(xl, src, dst, ex)
    den = jnp.pad(den, ((0, 0), (0, NPAD - NSC)))
    return num, den


def kernel(x, edge_index, Wl1, bl1, Wr1, br1, att1, bias1, Wlin1, blin1,
           Wl2, bl2, Wr2, br2, att2, bias2, Wlin2, blin2):
    f32 = jnp.float32
    xp = jnp.zeros((NPAD, FIN), f32).at[:N].set(x)
    loop = jnp.arange(N, dtype=jnp.int32)
    src = jnp.concatenate(
        [edge_index[0], loop, jnp.zeros((EPAD - E1,), jnp.int32)])
    dst = jnp.concatenate(
        [edge_index[1], loop, jnp.full((EPAD - E1,), DUMMY, jnp.int32)])

    def padw(wmat):
        return jnp.zeros((H, H), f32).at[:, :C].set(wmat)

    def padv(vec):
        return jnp.zeros((H,), f32).at[:C].set(vec)

    xl1, xr1, lin1 = _mm3(xp, Wl1, bl1, Wr1, br1, Wlin1, blin1)
    num1, den1 = _edge_softmax_scatter(xl1, xr1, src, dst, att1)
    xl2, xr2, lin2 = _tc_mid(num1, den1, lin1, bias1.reshape(1, H),
                             padw(Wl2), padv(bl2).reshape(1, H),
                             padw(Wr2), padv(br2).reshape(1, H),
                             padw(Wlin2), padv(blin2).reshape(1, H))
    num2, den2 = _edge_softmax_scatter(xl2, xr2, src, dst, padv(att2))
    o = _tc_post(num2, den2, lin2, padv(bias2).reshape(1, H))
    return (o[:N, :C], edge_index)
